# baseline (speedup 1.0000x reference)
import functools
import jax
import jax.numpy as jnp
from jax import lax
from jax.experimental import pallas as pl
from jax.experimental.pallas import tpu as pltpu
from jax.experimental.pallas import tpu_sc as plsc

CH = 10
CP = CH // 2
PH = 7
PW = 7
H = 34
W = 34
NBINS = PH * PW
STRIDE = 8.0
SPP = 4
N_ROIS = 5000

NC = 2
NS = 16
NW = NC * NS
L = 16
R_PER_TILE = 160
G_PER_TILE = R_PER_TILE // L
N_PAD = NW * R_PER_TILE
HWP = 1160
PLANE = CP * HWP
OUT_PER_TILE = R_PER_TILE * CH * NBINS


def _sc_body(ftp_hbm, rois_hbm, out_hbm, plane_a, plane_b, rois_v, params_v,
             acc_v, sem):
    wid = lax.axis_index("s") * NC + lax.axis_index("c")
    base = wid * R_PER_TILE

    for r in range(4):
        pltpu.sync_copy(
            rois_hbm.at[pl.ds(r * N_PAD + base, R_PER_TILE)],
            rois_v.at[pl.ds(r * R_PER_TILE, R_PER_TILE)],
        )

    for g in range(G_PER_TILE):
        def rv(r):
            return rois_v[pl.ds(r * R_PER_TILE + g * L, L)]
        rsw = rv(0) * (1.0 / STRIDE)
        rsh = rv(1) * (1.0 / STRIDE)
        rew = rv(2) * (1.0 / STRIDE)
        reh = rv(3) * (1.0 / STRIDE)
        rh = jnp.maximum(reh - rsh, 0.1)
        rw = jnp.maximum(rew - rsw, 0.1)
        bsh = rh * (1.0 / PH)
        bsw = rw * (1.0 / PW)
        for row, val in enumerate(
            (rsw, rsh, bsw, bsh, bsw * (1.0 / SPP), bsh * (1.0 / SPP))
        ):
            params_v[pl.ds(row * R_PER_TILE + g * L, L)] = val

    iota = lax.iota(jnp.int32, L)
    out_base_iota = iota * (CH * NBINS)
    himask = jnp.full((L,), -65536, jnp.int32)

    def unpack2(word):
        lo = lax.bitcast_convert_type(lax.shift_left(word, 16), jnp.float32)
        hi = lax.bitcast_convert_type(word & himask, jnp.float32)
        return lo, hi

    def compute_bin(b, plane):
        ph = b // PW
        pw = b - ph * PW
        phf = ph.astype(jnp.float32)
        pwf = pw.astype(jnp.float32)
        prefs = [plane.at[pl.ds(k * HWP, H * W)] for k in range(CP)]

        def one_group(g):
            gl = g * L

            def pv(row):
                return params_v[pl.ds(row * R_PER_TILE + gl, L)]
            rsw = pv(0)
            rsh = pv(1)
            bsw = pv(2)
            bsh = pv(3)
            ssw = pv(4)
            ssh = pv(5)
            wstart = (rsw + pwf * bsw).astype(jnp.int32).astype(jnp.float32)
            hstart = (rsh + phf * bsh).astype(jnp.int32).astype(jnp.float32)

            pk = functools.partial(
                plsc.pack, format=plsc.PackFormat.INTERLEAVED)
            xs = []
            nwv = jnp.zeros((L,), jnp.float32)
            for iw in range(SPP):
                ww = wstart + (iw + 0.5) * ssw
                x1i = ww.astype(jnp.int32)
                fx = ww - x1i.astype(jnp.float32)
                x2i = x1i + jnp.where(fx > 0.0, 1, 0)
                kw = ww < float(W)
                xbad = ~((x1i < W) & (x2i < W))
                x1c = jnp.minimum(x1i, W - 1)
                dxi = jnp.minimum(x2i, W - 1) - x1c
                fx_m = jnp.where(kw, fx, 0.0)
                omdx_m = jnp.where(kw, 1.0 - fx, 0.0)
                omdx11 = jnp.where(xbad, 0.0, omdx_m)
                nwv = nwv + jnp.where(kw, 1.0, 0.0)
                xs.append((x1c, dxi, pk(fx_m, fx_m), pk(omdx_m, omdx_m),
                           pk(omdx11, omdx11)))

            accs = [jnp.zeros((L,), jnp.float32) for _ in range(CH)]
            nhv = jnp.zeros((L,), jnp.float32)
            for ih in range(SPP):
                hh = hstart + (ih + 0.5) * ssh
                y1i = hh.astype(jnp.int32)
                fy = hh - y1i.astype(jnp.float32)
                y2i = y1i + jnp.where(fy > 0.0, 1, 0)
                kh = hh < float(H)
                y1r = jnp.minimum(y1i, H - 1) * W
                y2r = jnp.minimum(y2i, H - 1) * W
                fy_m = jnp.where(kh, fy, 0.0)
                omdy_m = jnp.where(kh, 1.0 - fy, 0.0)
                fyp = pk(fy_m, fy_m)
                omdyp = pk(omdy_m, omdy_m)
                nhv = nhv + jnp.where(kh, 1.0, 0.0)
                spairs = [jnp.zeros((2 * L,), jnp.bfloat16) for _ in range(CP)]
                for iw in range(SPP):
                    x1c, dxi, fxp, omdxp, omdx11p = xs[iw]
                    i11 = y1r + x1c
                    i12 = y2r + x1c
                    i21 = i11 + dxi
                    i22 = i12 + dxi
                    w11p = omdx11p * omdyp
                    w12p = omdxp * fyp
                    w21p = fxp * omdyp
                    w22p = fxp * fyp
                    for k in range(CP):
                        p11 = plsc.bitcast(
                            plsc.load_gather(prefs[k], [i11]), jnp.bfloat16)
                        p12 = plsc.bitcast(
                            plsc.load_gather(prefs[k], [i12]), jnp.bfloat16)
                        p21 = plsc.bitcast(
                            plsc.load_gather(prefs[k], [i21]), jnp.bfloat16)
                        p22 = plsc.bitcast(
                            plsc.load_gather(prefs[k], [i22]), jnp.bfloat16)
                        spairs[k] = spairs[k] + (
                            (w11p * p11 + w12p * p12)
                            + (w21p * p21 + w22p * p22))
                for k in range(CP):
                    lo, hi = unpack2(plsc.bitcast(spairs[k], jnp.int32))
                    accs[2 * k] = accs[2 * k] + lo
                    accs[2 * k + 1] = accs[2 * k + 1] + hi

            inv = 1.0 / jnp.maximum(nhv * nwv, 1.0)
            obase = g * (L * CH * NBINS) + b
            for c in range(CH):
                idxv = out_base_iota + (obase + c * NBINS)
                plsc.store_scatter(acc_v, [idxv], accs[c] * inv)

        def g_body(t, carry2):
            one_group(2 * t)
            one_group(2 * t + 1)
            return carry2

        lax.fori_loop(0, G_PER_TILE // 2, g_body, 0)

    def copy_plane(b, dst, sem_):
        return pltpu.make_async_copy(
            ftp_hbm.at[pl.ds(b * PLANE, PLANE)], dst, sem_
        )

    pltpu.async_copy(ftp_hbm.at[pl.ds(0, PLANE)], plane_a, sem)

    def pair_body(t, carry):
        b0 = 2 * t
        copy_plane(b0, plane_a, sem).wait()
        copy_plane(b0 + 1, plane_b, sem).start()
        compute_bin(b0, plane_a)
        copy_plane(b0 + 1, plane_b, sem).wait()
        copy_plane(b0 + 2, plane_a, sem).start()
        compute_bin(b0 + 1, plane_b)
        return carry

    lax.fori_loop(0, (NBINS - 1) // 2, pair_body, 0)
    copy_plane(NBINS - 1, plane_a, sem).wait()
    compute_bin(jnp.int32(NBINS - 1), plane_a)

    pltpu.sync_copy(acc_v, out_hbm.at[pl.ds(wid * OUT_PER_TILE, OUT_PER_TILE)])


@jax.jit
def _run(ftp, roisT):
    f = functools.partial(
        pl.kernel,
        out_type=jax.ShapeDtypeStruct((N_PAD * CH * NBINS,), jnp.float32),
        mesh=plsc.VectorSubcoreMesh(
            core_axis_name="c", subcore_axis_name="s",
            num_cores=NC, num_subcores=NS,
        ),
        scratch_types=[
            pltpu.VMEM((PLANE,), jnp.int32),
            pltpu.VMEM((PLANE,), jnp.int32),
            pltpu.VMEM((4 * R_PER_TILE,), jnp.float32),
            pltpu.VMEM((6 * R_PER_TILE,), jnp.float32),
            pltpu.VMEM((OUT_PER_TILE,), jnp.float32),
            pltpu.SemaphoreType.DMA,
        ],
        compiler_params=pltpu.CompilerParams(needs_layout_passes=False),
    )(_sc_body)
    return f(ftp, roisT)


def kernel(ft_add_left_right, rois):
    ftt = jnp.transpose(
        ft_add_left_right[0].reshape(CH, NBINS, H * W), (1, 0, 2)
    )
    u16 = lax.bitcast_convert_type(
        ftt.astype(jnp.bfloat16), jnp.uint16
    ).astype(jnp.uint32)
    packed = (u16[:, 0::2, :] | (u16[:, 1::2, :] << 16)).astype(jnp.int32)
    ftp = jnp.pad(packed, ((0, 0), (0, 0), (0, HWP - H * W))).reshape(
        NBINS * PLANE)
    roisT = jnp.pad(
        jnp.transpose(rois[:, 1:5]), ((0, 0), (0, N_PAD - N_ROIS))
    ).reshape(4 * N_PAD)
    out = _run(ftp, roisT)
    return out.reshape(N_PAD, CH, NBINS)[:N_ROIS]


# speedup vs baseline: 1.0454x; 1.0454x over previous
import functools
import jax
import jax.numpy as jnp
from jax import lax
from jax.experimental import pallas as pl
from jax.experimental.pallas import tpu as pltpu
from jax.experimental.pallas import tpu_sc as plsc

CH = 10
CP = CH // 2
PH = 7
PW = 7
H = 34
W = 34
NBINS = PH * PW
STRIDE = 8.0
SPP = 4
N_ROIS = 5000

NC = 2
NS = 16
NW = NC * NS
L = 16
R_PER_TILE = 160
G_PER_TILE = R_PER_TILE // L
N_PAD = NW * R_PER_TILE
HWP = 1160
PLANE = CP * HWP
OUT_PER_TILE = R_PER_TILE * CH * NBINS


def _sc_body(ftp_hbm, rois_hbm, out_hbm, plane_a, plane_b, rois_v, params_v,
             acc_v, sem):
    wid = lax.axis_index("s") * NC + lax.axis_index("c")
    base = wid * R_PER_TILE

    for r in range(4):
        pltpu.sync_copy(
            rois_hbm.at[pl.ds(r * N_PAD + base, R_PER_TILE)],
            rois_v.at[pl.ds(r * R_PER_TILE, R_PER_TILE)],
        )

    for g in range(G_PER_TILE):
        def rv(r):
            return rois_v[pl.ds(r * R_PER_TILE + g * L, L)]
        rsw = rv(0) * (1.0 / STRIDE)
        rsh = rv(1) * (1.0 / STRIDE)
        rew = rv(2) * (1.0 / STRIDE)
        reh = rv(3) * (1.0 / STRIDE)
        rh = jnp.maximum(reh - rsh, 0.1)
        rw = jnp.maximum(rew - rsw, 0.1)
        bsh = rh * (1.0 / PH)
        bsw = rw * (1.0 / PW)
        for row, val in enumerate(
            (rsw, rsh, bsw, bsh, bsw * (1.0 / SPP), bsh * (1.0 / SPP))
        ):
            params_v[pl.ds(row * R_PER_TILE + g * L, L)] = val

    iota = lax.iota(jnp.int32, L)
    out_base_iota = iota * (CH * NBINS)
    himask = jnp.full((L,), -65536, jnp.int32)

    def unpack2(word):
        lo = lax.bitcast_convert_type(lax.shift_left(word, 16), jnp.float32)
        hi = lax.bitcast_convert_type(word & himask, jnp.float32)
        return lo, hi

    def compute_bin(b, plane):
        ph = b // PW
        pw = b - ph * PW
        phf = ph.astype(jnp.float32)
        pwf = pw.astype(jnp.float32)
        prefs = [plane.at[pl.ds(k * HWP, H * W)] for k in range(CP)]

        def g_body(g, carry2):
            gl = g * L

            def pv(row):
                return params_v[pl.ds(row * R_PER_TILE + gl, L)]
            rsw = pv(0)
            rsh = pv(1)
            bsw = pv(2)
            bsh = pv(3)
            ssw = pv(4)
            ssh = pv(5)
            wstart = (rsw + pwf * bsw).astype(jnp.int32).astype(jnp.float32)
            hstart = (rsh + phf * bsh).astype(jnp.int32).astype(jnp.float32)

            pk = functools.partial(
                plsc.pack, format=plsc.PackFormat.INTERLEAVED)
            xs = []
            nwv = jnp.zeros((L,), jnp.float32)
            for iw in range(SPP):
                ww = wstart + (iw + 0.5) * ssw
                x1i = ww.astype(jnp.int32)
                fx = ww - x1i.astype(jnp.float32)
                x2i = x1i + jnp.where(fx > 0.0, 1, 0)
                kw = ww < float(W)
                xbad = ~((x1i < W) & (x2i < W))
                x1c = jnp.minimum(x1i, W - 1)
                dxi = jnp.minimum(x2i, W - 1) - x1c
                fx_m = jnp.where(kw, fx, 0.0)
                omdx_m = jnp.where(kw, 1.0 - fx, 0.0)
                omdx11 = jnp.where(xbad, 0.0, omdx_m)
                nwv = nwv + jnp.where(kw, 1.0, 0.0)
                xs.append((x1c, dxi, pk(fx_m, fx_m), pk(omdx_m, omdx_m),
                           pk(omdx11, omdx11)))

            accs = [jnp.zeros((L,), jnp.float32) for _ in range(CH)]
            nhv = jnp.zeros((L,), jnp.float32)
            for ih in range(SPP):
                hh = hstart + (ih + 0.5) * ssh
                y1i = hh.astype(jnp.int32)
                fy = hh - y1i.astype(jnp.float32)
                y2i = y1i + jnp.where(fy > 0.0, 1, 0)
                kh = hh < float(H)
                y1r = jnp.minimum(y1i, H - 1) * W
                y2r = jnp.minimum(y2i, H - 1) * W
                fy_m = jnp.where(kh, fy, 0.0)
                omdy_m = jnp.where(kh, 1.0 - fy, 0.0)
                fyp = pk(fy_m, fy_m)
                omdyp = pk(omdy_m, omdy_m)
                nhv = nhv + jnp.where(kh, 1.0, 0.0)
                spairs = [jnp.zeros((2 * L,), jnp.bfloat16) for _ in range(CP)]
                for iw in range(SPP):
                    x1c, dxi, fxp, omdxp, omdx11p = xs[iw]
                    i11 = y1r + x1c
                    i12 = y2r + x1c
                    i21 = i11 + dxi
                    i22 = i12 + dxi
                    w11p = omdx11p * omdyp
                    w12p = omdxp * fyp
                    w21p = fxp * omdyp
                    w22p = fxp * fyp
                    for k in range(CP):
                        p11 = plsc.bitcast(
                            plsc.load_gather(prefs[k], [i11]), jnp.bfloat16)
                        p12 = plsc.bitcast(
                            plsc.load_gather(prefs[k], [i12]), jnp.bfloat16)
                        p21 = plsc.bitcast(
                            plsc.load_gather(prefs[k], [i21]), jnp.bfloat16)
                        p22 = plsc.bitcast(
                            plsc.load_gather(prefs[k], [i22]), jnp.bfloat16)
                        spairs[k] = spairs[k] + (
                            (w11p * p11 + w12p * p12)
                            + (w21p * p21 + w22p * p22))
                for k in range(CP):
                    lo, hi = unpack2(plsc.bitcast(spairs[k], jnp.int32))
                    accs[2 * k] = accs[2 * k] + lo
                    accs[2 * k + 1] = accs[2 * k + 1] + hi

            inv = 1.0 / jnp.maximum(nhv * nwv, 1.0)
            obase = g * (L * CH * NBINS) + b
            for c in range(CH):
                idxv = out_base_iota + (obase + c * NBINS)
                plsc.store_scatter(acc_v, [idxv], accs[c] * inv)
            return carry2

        lax.fori_loop(0, G_PER_TILE, g_body, 0)

    def copy_plane(b, dst, sem_):
        return pltpu.make_async_copy(
            ftp_hbm.at[pl.ds(b * PLANE, PLANE)], dst, sem_
        )

    pltpu.async_copy(ftp_hbm.at[pl.ds(0, PLANE)], plane_a, sem)

    def pair_body(t, carry):
        b0 = 2 * t
        copy_plane(b0, plane_a, sem).wait()
        copy_plane(b0 + 1, plane_b, sem).start()
        compute_bin(b0, plane_a)
        copy_plane(b0 + 1, plane_b, sem).wait()
        copy_plane(b0 + 2, plane_a, sem).start()
        compute_bin(b0 + 1, plane_b)
        return carry

    lax.fori_loop(0, (NBINS - 1) // 2, pair_body, 0)
    copy_plane(NBINS - 1, plane_a, sem).wait()
    compute_bin(jnp.int32(NBINS - 1), plane_a)

    pltpu.sync_copy(acc_v, out_hbm.at[pl.ds(wid * OUT_PER_TILE, OUT_PER_TILE)])


@jax.jit
def _run(ftp, roisT):
    f = functools.partial(
        pl.kernel,
        out_type=jax.ShapeDtypeStruct((N_PAD * CH * NBINS,), jnp.float32),
        mesh=plsc.VectorSubcoreMesh(
            core_axis_name="c", subcore_axis_name="s",
            num_cores=NC, num_subcores=NS,
        ),
        scratch_types=[
            pltpu.VMEM((PLANE,), jnp.int32),
            pltpu.VMEM((PLANE,), jnp.int32),
            pltpu.VMEM((4 * R_PER_TILE,), jnp.float32),
            pltpu.VMEM((6 * R_PER_TILE,), jnp.float32),
            pltpu.VMEM((OUT_PER_TILE,), jnp.float32),
            pltpu.SemaphoreType.DMA,
        ],
        compiler_params=pltpu.CompilerParams(needs_layout_passes=False),
    )(_sc_body)
    return f(ftp, roisT)


def kernel(ft_add_left_right, rois):
    ftt = jnp.transpose(
        ft_add_left_right[0].reshape(CH, NBINS, H * W), (1, 0, 2)
    )
    u16 = lax.bitcast_convert_type(
        ftt.astype(jnp.bfloat16), jnp.uint16
    ).astype(jnp.uint32)
    packed = (u16[:, 0::2, :] | (u16[:, 1::2, :] << 16)).astype(jnp.int32)
    ftp = jnp.pad(packed, ((0, 0), (0, 0), (0, HWP - H * W))).reshape(
        NBINS * PLANE)
    roisT = jnp.pad(
        jnp.transpose(rois[:, 1:5]), ((0, 0), (0, N_PAD - N_ROIS))
    ).reshape(4 * N_PAD)
    out = _run(ftp, roisT)
    return out.reshape(N_PAD, CH, NBINS)[:N_ROIS]


# speedup vs baseline: 1.0589x; 1.0129x over previous
import functools
import jax
import jax.numpy as jnp
from jax import lax
from jax.experimental import pallas as pl
from jax.experimental.pallas import tpu as pltpu
from jax.experimental.pallas import tpu_sc as plsc

CH = 10
CP = CH // 2
PH = 7
PW = 7
H = 34
W = 34
NBINS = PH * PW
STRIDE = 8.0
SPP = 4
N_ROIS = 5000

NC = 2
NS = 16
NW = NC * NS
L = 16
R_PER_TILE = 160
G_PER_TILE = R_PER_TILE // L
N_PAD = NW * R_PER_TILE
HWP = 1160
PLANE = CP * HWP
OUT_PER_TILE = R_PER_TILE * CH * NBINS
LAST_TILE_OUT = (N_ROIS - (NW - 1) * R_PER_TILE) * CH * NBINS


def _sc_body(ftp_hbm, rois_hbm, out_hbm, plane_a, plane_b, rois_v, params_v,
             acc_v, sem):
    wid = lax.axis_index("s") * NC + lax.axis_index("c")
    base = wid * R_PER_TILE

    iota16 = lax.iota(jnp.int32, L)

    pltpu.sync_copy(
        rois_hbm.at[pl.ds(base * 5, R_PER_TILE * 5)], rois_v)

    iota5 = iota16 * 5
    for g in range(G_PER_TILE):
        def rv(r):
            return plsc.load_gather(rois_v, [iota5 + (g * L * 5 + 1 + r)])
        rsw = rv(0) * (1.0 / STRIDE)
        rsh = rv(1) * (1.0 / STRIDE)
        rew = rv(2) * (1.0 / STRIDE)
        reh = rv(3) * (1.0 / STRIDE)
        rh = jnp.maximum(reh - rsh, 0.1)
        rw = jnp.maximum(rew - rsw, 0.1)
        bsh = rh * (1.0 / PH)
        bsw = rw * (1.0 / PW)
        for row, val in enumerate(
            (rsw, rsh, bsw, bsh, bsw * (1.0 / SPP), bsh * (1.0 / SPP))
        ):
            params_v[pl.ds(row * R_PER_TILE + g * L, L)] = val

    out_base_iota = iota16 * (CH * NBINS)
    himask = jnp.full((L,), -65536, jnp.int32)

    def unpack2(word):
        lo = lax.bitcast_convert_type(lax.shift_left(word, 16), jnp.float32)
        hi = lax.bitcast_convert_type(word & himask, jnp.float32)
        return lo, hi

    def compute_bin(b, plane):
        ph = b // PW
        pw = b - ph * PW
        phf = ph.astype(jnp.float32)
        pwf = pw.astype(jnp.float32)
        prefs = [plane.at[pl.ds(k * HWP, H * W)] for k in range(CP)]

        def g_body(g, carry2):
            gl = g * L

            def pv(row):
                return params_v[pl.ds(row * R_PER_TILE + gl, L)]
            rsw = pv(0)
            rsh = pv(1)
            bsw = pv(2)
            bsh = pv(3)
            ssw = pv(4)
            ssh = pv(5)
            wstart = (rsw + pwf * bsw).astype(jnp.int32).astype(jnp.float32)
            hstart = (rsh + phf * bsh).astype(jnp.int32).astype(jnp.float32)

            pk = functools.partial(
                plsc.pack, format=plsc.PackFormat.INTERLEAVED)
            xs = []
            nwv = jnp.zeros((L,), jnp.float32)
            for iw in range(SPP):
                ww = wstart + (iw + 0.5) * ssw
                x1i = ww.astype(jnp.int32)
                fx = ww - x1i.astype(jnp.float32)
                x2i = x1i + jnp.where(fx > 0.0, 1, 0)
                kw = ww < float(W)
                xbad = ~((x1i < W) & (x2i < W))
                x1c = jnp.minimum(x1i, W - 1)
                dxi = jnp.minimum(x2i, W - 1) - x1c
                fx_m = jnp.where(kw, fx, 0.0)
                omdx_m = jnp.where(kw, 1.0 - fx, 0.0)
                omdx11 = jnp.where(xbad, 0.0, omdx_m)
                nwv = nwv + jnp.where(kw, 1.0, 0.0)
                xs.append((x1c, dxi, pk(fx_m, fx_m), pk(omdx_m, omdx_m),
                           pk(omdx11, omdx11)))

            accs = [jnp.zeros((L,), jnp.float32) for _ in range(CH)]
            nhv = jnp.zeros((L,), jnp.float32)
            for ih in range(SPP):
                hh = hstart + (ih + 0.5) * ssh
                y1i = hh.astype(jnp.int32)
                fy = hh - y1i.astype(jnp.float32)
                y2i = y1i + jnp.where(fy > 0.0, 1, 0)
                kh = hh < float(H)
                y1r = jnp.minimum(y1i, H - 1) * W
                y2r = jnp.minimum(y2i, H - 1) * W
                fy_m = jnp.where(kh, fy, 0.0)
                omdy_m = jnp.where(kh, 1.0 - fy, 0.0)
                fyp = pk(fy_m, fy_m)
                omdyp = pk(omdy_m, omdy_m)
                nhv = nhv + jnp.where(kh, 1.0, 0.0)
                spairs = [jnp.zeros((2 * L,), jnp.bfloat16) for _ in range(CP)]
                for iw in range(SPP):
                    x1c, dxi, fxp, omdxp, omdx11p = xs[iw]
                    i11 = y1r + x1c
                    i12 = y2r + x1c
                    i21 = i11 + dxi
                    i22 = i12 + dxi
                    w11p = omdx11p * omdyp
                    w12p = omdxp * fyp
                    w21p = fxp * omdyp
                    w22p = fxp * fyp
                    for k in range(CP):
                        p11 = plsc.bitcast(
                            plsc.load_gather(prefs[k], [i11]), jnp.bfloat16)
                        p12 = plsc.bitcast(
                            plsc.load_gather(prefs[k], [i12]), jnp.bfloat16)
                        p21 = plsc.bitcast(
                            plsc.load_gather(prefs[k], [i21]), jnp.bfloat16)
                        p22 = plsc.bitcast(
                            plsc.load_gather(prefs[k], [i22]), jnp.bfloat16)
                        spairs[k] = spairs[k] + (
                            (w11p * p11 + w12p * p12)
                            + (w21p * p21 + w22p * p22))
                for k in range(CP):
                    lo, hi = unpack2(plsc.bitcast(spairs[k], jnp.int32))
                    accs[2 * k] = accs[2 * k] + lo
                    accs[2 * k + 1] = accs[2 * k + 1] + hi

            inv = 1.0 / jnp.maximum(nhv * nwv, 1.0)
            obase = g * (L * CH * NBINS) + b
            for c in range(CH):
                idxv = out_base_iota + (obase + c * NBINS)
                plsc.store_scatter(acc_v, [idxv], accs[c] * inv)
            return carry2

        lax.fori_loop(0, G_PER_TILE, g_body, 0)

    def plane_copies(b, dst):
        return [
            pltpu.make_async_copy(
                ftp_hbm.at[pl.ds((k * NBINS + b) * HWP, HWP)],
                dst.at[pl.ds(k * HWP, HWP)],
                sem,
            )
            for k in range(CP)
        ]

    def start_plane(b, dst):
        for cp in plane_copies(b, dst):
            cp.start()

    def wait_plane(b, dst):
        for cp in plane_copies(b, dst):
            cp.wait()

    start_plane(0, plane_a)

    def pair_body(t, carry):
        b0 = 2 * t
        wait_plane(b0, plane_a)
        start_plane(b0 + 1, plane_b)
        compute_bin(b0, plane_a)
        wait_plane(b0 + 1, plane_b)
        start_plane(b0 + 2, plane_a)
        compute_bin(b0 + 1, plane_b)
        return carry

    lax.fori_loop(0, (NBINS - 1) // 2, pair_body, 0)
    wait_plane(NBINS - 1, plane_a)
    compute_bin(jnp.int32(NBINS - 1), plane_a)

    @pl.when(wid < NW - 1)
    def _full():
        pltpu.sync_copy(
            acc_v, out_hbm.at[pl.ds(wid * OUT_PER_TILE, OUT_PER_TILE)])

    @pl.when(wid == NW - 1)
    def _tail():
        pltpu.sync_copy(
            acc_v.at[pl.ds(0, LAST_TILE_OUT)],
            out_hbm.at[pl.ds((NW - 1) * OUT_PER_TILE, LAST_TILE_OUT)])


@jax.jit
def _run(ftp, rois_flat):
    f = functools.partial(
        pl.kernel,
        out_type=jax.ShapeDtypeStruct((N_ROIS * CH * NBINS,), jnp.float32),
        mesh=plsc.VectorSubcoreMesh(
            core_axis_name="c", subcore_axis_name="s",
            num_cores=NC, num_subcores=NS,
        ),
        scratch_types=[
            pltpu.VMEM((PLANE,), jnp.int32),
            pltpu.VMEM((PLANE,), jnp.int32),
            pltpu.VMEM((5 * R_PER_TILE,), jnp.float32),
            pltpu.VMEM((6 * R_PER_TILE,), jnp.float32),
            pltpu.VMEM((OUT_PER_TILE,), jnp.float32),
            pltpu.SemaphoreType.DMA,
        ],
        compiler_params=pltpu.CompilerParams(needs_layout_passes=False),
    )(_sc_body)
    return f(ftp, rois_flat)


def kernel(ft_add_left_right, rois):
    u16 = lax.bitcast_convert_type(
        ft_add_left_right[0].reshape(CH, NBINS, H * W).astype(jnp.bfloat16),
        jnp.uint16,
    ).astype(jnp.uint32)
    packed = (u16[0::2] | (u16[1::2] << 16)).astype(jnp.int32)
    ftp = jnp.pad(packed, ((0, 0), (0, 0), (0, HWP - H * W))).reshape(
        CP * NBINS * HWP)
    rois_flat = jnp.pad(rois, ((0, N_PAD - N_ROIS), (0, 0))).reshape(N_PAD * 5)
    out = _run(ftp, rois_flat)
    return out.reshape(N_ROIS, CH, NBINS)
